# baseline (speedup 1.0000x reference)
import dataclasses
import functools

import jax
import jax.numpy as jnp
from jax import lax
from jax.experimental import pallas as pl
from jax.experimental.pallas import tpu as pltpu
from jax.experimental.pallas import tpu_sc as plsc

_SPARSITY = 0.1


def _stage_a_body(F, x_ref, w1t_ref, b1_ref, w2t_ref, a1_ref, c1_ref,
                  a2_ref, c2_ref, a3_ref,
                  keys_ref, u_ref, vf_ref, b2_ref, c2t_ref, a3s_ref):
    x = x_ref[...]
    h = jax.nn.relu(jnp.dot(x, w1t_ref[...]) + b1_ref[...])
    sc = jnp.dot(h, w2t_ref[...])
    b = jax.lax.bitcast_convert_type(sc, jnp.int32)
    neg = jax.lax.shift_right_arithmetic(b, 31)
    ukey = b ^ (neg | jnp.int32(-2147483648))
    n = ukey.shape[0]
    keys_ref[...] = ukey.reshape(n // 128, 128)
    a1 = a1_ref[...]
    hiprec = jax.lax.Precision.HIGHEST
    u_ref[...] = jax.lax.dot_general(
        x, a1[:, :F], (((1,), (1,)), ((), ())),
        precision=hiprec)
    a1kv = a1[:, F:2 * F] + a1[:, 2 * F:]
    vf = jax.lax.dot_general(
        x, a1kv, (((1,), (1,)), ((), ())),
        precision=hiprec) + c1_ref[...]
    vf_ref[...] = jnp.concatenate(
        [vf, jnp.zeros_like(vf)], axis=1)
    a2t = jnp.transpose(a2_ref[...])
    z = jnp.zeros_like(a2t)
    rows = [jnp.concatenate([a2t if q == p else z for q in range(4)],
                            axis=1) for p in range(4)]
    b2_ref[...] = jnp.concatenate(rows, axis=0)
    c2r = c2_ref[...]
    c2t_ref[...] = jnp.concatenate([c2r] * 4, axis=1)
    a3t = jnp.transpose(a3_ref[...])
    a3s_ref[...] = jnp.concatenate([a3t] * 4, axis=0)


def _run_stage_a(x, W1T, b1, W2T, A1, c1, A2, c2, A3):
    n, F = x.shape
    return pl.pallas_call(
        functools.partial(_stage_a_body, F),
        out_shape=(
            jax.ShapeDtypeStruct((n // 128, 128), jnp.int32),
            jax.ShapeDtypeStruct((n, 64), jnp.float32),
            jax.ShapeDtypeStruct((n, 128), jnp.float32),
            jax.ShapeDtypeStruct((256, 128), jnp.float32),
            jax.ShapeDtypeStruct((1, 128), jnp.float32),
            jax.ShapeDtypeStruct((128, F), jnp.float32),
        ),
    )(x, W1T, b1, W2T, A1, c1, A2, c2, A3)


def _make_sc_select(S, K, KP):
    n_vregs = S // 16
    n_rows = KP // 16
    mesh = plsc.VectorSubcoreMesh(core_axis_name="c", subcore_axis_name="s")

    def body(keys_hbm, vf_hbm, out_hbm, key_v, idx_v, vbuf, hist_v, sem):
        cid = lax.axis_index("c")
        sid = lax.axis_index("s")

        @pl.when(sid == 0)
        def _():
            base = cid * S
            pltpu.sync_copy(keys_hbm.at[pl.ds(base, S)], key_v)

            iot = lax.iota(jnp.int32, 16)
            ones = jnp.zeros((16,), jnp.int32) + 1

            prefix = jnp.int32(0)
            k_rem = jnp.int32(K)
            for p in range(4):
                shift = 24 - 8 * p
                for r in range(16):
                    hist_v[r] = jnp.zeros((16,), jnp.int32)

                pfx = prefix

                @pl.loop(0, n_vregs)
                def _(i, shift=shift, p=p, pfx=pfx):
                    kk = key_v[pl.ds(i * 16, 16)]
                    byte = lax.shift_right_logical(kk, shift) & jnp.int32(255)
                    row = lax.shift_right_logical(byte, 4)
                    col = byte & jnp.int32(15)
                    if p == 0:
                        plsc.addupdate_scatter(hist_v, [row, col], ones)
                    else:
                        act = lax.shift_right_logical(
                            kk, shift + 8) == pfx
                        plsc.addupdate_scatter(hist_v, [row, col], ones,
                                               mask=act)

                carry = jnp.int32(0)
                best = jnp.int32(0)
                for r in range(15, -1, -1):
                    h = hist_v[r]
                    tot = jnp.sum(h)
                    incl = plsc.cumsum(h)
                    ge = (carry + tot) - incl + h
                    cand = jnp.max(jnp.where(ge >= k_rem,
                                             iot + 16 * r, 0))
                    best = jnp.maximum(best, cand)
                    carry = carry + tot
                bsel = best
                accv = jnp.zeros((16,), jnp.int32)
                for r in range(16):
                    accv = accv + jnp.where(iot + 16 * r > bsel,
                                            hist_v[r], 0)
                cnt_gt = jnp.sum(accv)
                prefix = prefix * 256 + bsel
                k_rem = k_rem - cnt_gt

            utarget = prefix
            need = k_rem
            ngt = K - need
            sgn = jnp.int32(-2147483648)
            tvx = utarget ^ sgn

            for r in range(n_rows):
                idx_v[r] = jnp.zeros((16,), jnp.int32) + base

            def sbody(i, packed):
                cgt = packed // 8192
                ceq = packed - cgt * 8192
                kk = key_v[pl.ds(i * 16, 16)]
                gt = (kk ^ sgn) > tvx
                eq = kk == utarget
                gti = jnp.where(gt, 1, 0).astype(jnp.int32)
                eqi = jnp.where(eq, 1, 0).astype(jnp.int32)
                pgt = plsc.cumsum(gti) + cgt
                peq = plsc.cumsum(eqi) + ceq
                slot = jnp.where(gt, pgt - 1, ngt + peq - 1)
                slot = jnp.minimum(slot, jnp.int32(KP - 1))
                slot = jnp.maximum(slot, jnp.int32(0))
                m = gt | (eq & (peq <= need))
                tok = lax.iota(jnp.int32, 16) + (i * 16 + base)
                row = lax.shift_right_arithmetic(slot, 4)
                col = slot & jnp.int32(15)
                plsc.store_scatter(idx_v, [row, col], tok, mask=m)
                return ((cgt + jnp.sum(gti)) * 8192 + (ceq + jnp.sum(eqi)))

            lax.fori_loop(0, n_vregs, sbody, jnp.int32(0))

            copies = []
            for r in range(n_rows):
                copies.append(pltpu.async_copy(
                    vf_hbm.at[idx_v.at[r]],
                    vbuf.at[pl.ds(r * 16, 16)], sem))
            for c in copies:
                c.wait()
            pltpu.sync_copy(vbuf, out_hbm.at[cid])

    cp = pltpu.CompilerParams()
    if "needs_layout_passes" in pltpu.CompilerParams.__dataclass_fields__:
        cp = dataclasses.replace(cp, needs_layout_passes=False)
    return functools.partial(
        pl.kernel, body,
        out_type=jax.ShapeDtypeStruct((2, KP, 128), jnp.float32),
        mesh=mesh,
        compiler_params=cp,
        scratch_types=[
            pltpu.VMEM((S,), jnp.int32),
            pltpu.VMEM((KP // 16, 16), jnp.int32),
            pltpu.VMEM((KP, 128), jnp.float32),
            pltpu.VMEM((16, 16), jnp.int32),
            pltpu.SemaphoreType.DMA,
        ],
    )()


def _stage_c_body(groups, inv_k, u_ref, v2_ref, b2_ref, c2_ref, a3_ref,
                  c3_ref, out_ref):
    u = u_ref[0]
    u4 = jnp.concatenate([u, u, u, u], axis=1)
    v52 = v2_ref[0]
    b2b = b2_ref[...].astype(jnp.bfloat16)
    c2t = c2_ref[...]
    ts = u.shape[0]
    acc = jnp.zeros((ts, 128), jnp.float32)
    for jg in range(groups):
        vrow = v52[jg:jg + 1, :]
        h = jax.nn.relu(u4 + vrow)
        g = jnp.dot(h.astype(jnp.bfloat16), b2b,
                    preferred_element_type=jnp.float32)
        acc = acc + jax.nn.relu(g + c2t)
    out = jax.lax.dot_general(
        acc * inv_k, a3_ref[...], (((1,), (0,)), ((), ())),
        precision=jax.lax.Precision.HIGHEST) + c3_ref[...]
    out_ref[0] = out


def _run_stage_c(u3, v2, b2, c2t, a3s, c3r, K, TS):
    B, S, _ = u3.shape
    G = v2.shape[1]
    body = functools.partial(_stage_c_body, G, 1.0 / K)
    return pl.pallas_call(
        body,
        grid=(B, S // TS),
        in_specs=[
            pl.BlockSpec((1, TS, 64), lambda b, i: (b, i, 0)),
            pl.BlockSpec((1, G, 256), lambda b, i: (b, 0, 0)),
            pl.BlockSpec((256, 128), lambda b, i: (0, 0)),
            pl.BlockSpec((1, 128), lambda b, i: (0, 0)),
            pl.BlockSpec((128, 10), lambda b, i: (0, 0)),
            pl.BlockSpec((1, 10), lambda b, i: (0, 0)),
        ],
        out_specs=pl.BlockSpec((1, TS, 10), lambda b, i: (b, i, 0)),
        out_shape=jax.ShapeDtypeStruct((B, S, 10), jnp.float32),
    )(u3, v2, b2, c2t, a3s, c3r)


def kernel(full, W1, b1, W2, b2, A1, c1, A2, c2, A3, c3):
    B, S, F = full.shape
    K = max(1, int(S * _SPARSITY))
    KP = ((K + 15) // 16) * 16
    G = K // 4

    x = full.reshape(B * S, F)
    keys, u, vf, b2blk, c2t, a3s = _run_stage_a(
        x, W1.T, b1.reshape(1, 32), W2.T, A1, c1.reshape(1, 64),
        A2, c2.reshape(1, 32), A3)

    sc_select = _make_sc_select(S, K, KP)
    v = sc_select(keys.reshape(B * S), vf)

    u3 = u.reshape(B, S, 64)
    v2 = v[:, :K, :64].reshape(B, G, 256)

    out = _run_stage_c(u3, v2, b2blk, c2t, a3s, c3.reshape(1, F), K, TS=512)
    return out


# speedup vs baseline: 1.0196x; 1.0196x over previous
import dataclasses
import functools

import jax
import jax.numpy as jnp
from jax import lax
from jax.experimental import pallas as pl
from jax.experimental.pallas import tpu as pltpu
from jax.experimental.pallas import tpu_sc as plsc

_SPARSITY = 0.1


def _stage_a_body(F, x_ref, w1t_ref, b1_ref, w2t_ref, a1_ref, c1_ref,
                  a2_ref, c2_ref, a3_ref,
                  keys_ref, u_ref, vf_ref, b2_ref, c2t_ref, a3s_ref):
    x = x_ref[...]
    h = jax.nn.relu(jnp.dot(x, w1t_ref[...]) + b1_ref[...])
    sc = jnp.dot(h, w2t_ref[...])
    b = jax.lax.bitcast_convert_type(sc, jnp.int32)
    neg = jax.lax.shift_right_arithmetic(b, 31)
    ukey = b ^ (neg | jnp.int32(-2147483648))
    n = ukey.shape[0]
    keys_ref[...] = ukey.reshape(n // 128, 128)
    a1 = a1_ref[...]
    hiprec = jax.lax.Precision.HIGHEST
    u_ref[...] = jax.lax.dot_general(
        x, a1[:, :F], (((1,), (1,)), ((), ())),
        precision=hiprec)
    a1kv = a1[:, F:2 * F] + a1[:, 2 * F:]
    vf = jax.lax.dot_general(
        x, a1kv, (((1,), (1,)), ((), ())),
        precision=hiprec) + c1_ref[...]
    vf_ref[...] = jnp.concatenate(
        [vf, jnp.zeros_like(vf)], axis=1)
    a2t = jnp.transpose(a2_ref[...])
    z = jnp.zeros_like(a2t)
    rows = [jnp.concatenate([a2t if q == p else z for q in range(4)],
                            axis=1) for p in range(4)]
    b2_ref[...] = jnp.concatenate(rows, axis=0).astype(jnp.bfloat16)
    c2r = c2_ref[...]
    c2t_ref[...] = jnp.concatenate([c2r] * 4, axis=1)
    a3t = jnp.transpose(a3_ref[...])
    a3s_ref[...] = jnp.concatenate([a3t] * 4, axis=0)


def _run_stage_a(x, W1T, b1, W2T, A1, c1, A2, c2, A3):
    n, F = x.shape
    return pl.pallas_call(
        functools.partial(_stage_a_body, F),
        out_shape=(
            jax.ShapeDtypeStruct((n // 128, 128), jnp.int32),
            jax.ShapeDtypeStruct((n, 64), jnp.float32),
            jax.ShapeDtypeStruct((n, 128), jnp.float32),
            jax.ShapeDtypeStruct((256, 128), jnp.bfloat16),
            jax.ShapeDtypeStruct((1, 128), jnp.float32),
            jax.ShapeDtypeStruct((128, F), jnp.float32),
        ),
    )(x, W1T, b1, W2T, A1, c1, A2, c2, A3)


def _make_sc_select(S, K, KP):
    n_vregs = S // 16
    n_rows = KP // 16
    mesh = plsc.VectorSubcoreMesh(core_axis_name="c", subcore_axis_name="s")

    def body(keys_hbm, vf_hbm, out_hbm, key_v, idx_v, vbuf, hist_v, sem):
        cid = lax.axis_index("c")
        sid = lax.axis_index("s")

        @pl.when(sid == 0)
        def _():
            base = cid * S
            pltpu.sync_copy(keys_hbm.at[pl.ds(base, S)], key_v)

            iot = lax.iota(jnp.int32, 16)
            ones = jnp.zeros((16,), jnp.int32) + 1

            prefix = jnp.int32(0)
            k_rem = jnp.int32(K)
            for p in range(4):
                shift = 24 - 8 * p
                for r in range(16):
                    hist_v[r] = jnp.zeros((16,), jnp.int32)

                pfx = prefix

                @pl.loop(0, n_vregs)
                def _(i, shift=shift, p=p, pfx=pfx):
                    kk = key_v[pl.ds(i * 16, 16)]
                    byte = lax.shift_right_logical(kk, shift) & jnp.int32(255)
                    row = lax.shift_right_logical(byte, 4)
                    col = byte & jnp.int32(15)
                    if p == 0:
                        plsc.addupdate_scatter(hist_v, [row, col], ones)
                    else:
                        act = lax.shift_right_logical(
                            kk, shift + 8) == pfx
                        plsc.addupdate_scatter(hist_v, [row, col], ones,
                                               mask=act)

                carry = jnp.int32(0)
                best = jnp.int32(0)
                for r in range(15, -1, -1):
                    h = hist_v[r]
                    tot = jnp.sum(h)
                    incl = plsc.cumsum(h)
                    ge = (carry + tot) - incl + h
                    cand = jnp.max(jnp.where(ge >= k_rem,
                                             iot + 16 * r, 0))
                    best = jnp.maximum(best, cand)
                    carry = carry + tot
                bsel = best
                accv = jnp.zeros((16,), jnp.int32)
                for r in range(16):
                    accv = accv + jnp.where(iot + 16 * r > bsel,
                                            hist_v[r], 0)
                cnt_gt = jnp.sum(accv)
                prefix = prefix * 256 + bsel
                k_rem = k_rem - cnt_gt

            utarget = prefix
            need = k_rem
            ngt = K - need
            sgn = jnp.int32(-2147483648)
            tvx = utarget ^ sgn

            for r in range(n_rows):
                idx_v[r] = jnp.zeros((16,), jnp.int32) + base

            def sbody(i, packed):
                cgt = packed // 8192
                ceq = packed - cgt * 8192
                kk = key_v[pl.ds(i * 16, 16)]
                gt = (kk ^ sgn) > tvx
                eq = kk == utarget
                gti = jnp.where(gt, 1, 0).astype(jnp.int32)
                eqi = jnp.where(eq, 1, 0).astype(jnp.int32)
                pgt = plsc.cumsum(gti) + cgt
                peq = plsc.cumsum(eqi) + ceq
                slot = jnp.where(gt, pgt - 1, ngt + peq - 1)
                slot = jnp.minimum(slot, jnp.int32(KP - 1))
                slot = jnp.maximum(slot, jnp.int32(0))
                m = gt | (eq & (peq <= need))
                tok = lax.iota(jnp.int32, 16) + (i * 16 + base)
                row = lax.shift_right_arithmetic(slot, 4)
                col = slot & jnp.int32(15)
                plsc.store_scatter(idx_v, [row, col], tok, mask=m)
                return ((cgt + jnp.sum(gti)) * 8192 + (ceq + jnp.sum(eqi)))

            lax.fori_loop(0, n_vregs, sbody, jnp.int32(0))

            copies = []
            for r in range(n_rows):
                copies.append(pltpu.async_copy(
                    vf_hbm.at[idx_v.at[r]],
                    vbuf.at[pl.ds(r * 16, 16)], sem))
            for c in copies:
                c.wait()
            pltpu.sync_copy(vbuf, out_hbm.at[cid])

    cp = pltpu.CompilerParams()
    if "needs_layout_passes" in pltpu.CompilerParams.__dataclass_fields__:
        cp = dataclasses.replace(cp, needs_layout_passes=False)
    return functools.partial(
        pl.kernel, body,
        out_type=jax.ShapeDtypeStruct((2, KP, 128), jnp.float32),
        mesh=mesh,
        compiler_params=cp,
        scratch_types=[
            pltpu.VMEM((S,), jnp.int32),
            pltpu.VMEM((KP // 16, 16), jnp.int32),
            pltpu.VMEM((KP, 128), jnp.float32),
            pltpu.VMEM((16, 16), jnp.int32),
            pltpu.SemaphoreType.DMA,
        ],
    )()


def _stage_c_body(groups, inv_k, u_ref, v2_ref, b2_ref, c2_ref, a3_ref,
                  c3_ref, out_ref):
    u = u_ref[0]
    u4 = jnp.concatenate([u, u, u, u], axis=1)
    v52 = v2_ref[0]
    b2b = b2_ref[...]
    c2t = c2_ref[...]
    ts = u.shape[0]
    acc = jnp.zeros((ts, 128), jnp.float32)
    for jg in range(groups):
        vrow = v52[jg:jg + 1, :]
        h = jax.nn.relu(u4 + vrow)
        g = jnp.dot(h.astype(jnp.bfloat16), b2b,
                    preferred_element_type=jnp.float32)
        acc = acc + jax.nn.relu(g + c2t)
    out = jax.lax.dot_general(
        acc * inv_k, a3_ref[...], (((1,), (0,)), ((), ())),
        precision=jax.lax.Precision.HIGHEST) + c3_ref[...]
    out_ref[0] = out


def _run_stage_c(u3, v2, b2, c2t, a3s, c3r, K, TS):
    B, S, _ = u3.shape
    G = v2.shape[1]
    body = functools.partial(_stage_c_body, G, 1.0 / K)
    return pl.pallas_call(
        body,
        grid=(B, S // TS),
        in_specs=[
            pl.BlockSpec((1, TS, 64), lambda b, i: (b, i, 0)),
            pl.BlockSpec((1, G, 256), lambda b, i: (b, 0, 0)),
            pl.BlockSpec((256, 128), lambda b, i: (0, 0)),
            pl.BlockSpec((1, 128), lambda b, i: (0, 0)),
            pl.BlockSpec((128, 10), lambda b, i: (0, 0)),
            pl.BlockSpec((1, 10), lambda b, i: (0, 0)),
        ],
        out_specs=pl.BlockSpec((1, TS, 10), lambda b, i: (b, i, 0)),
        out_shape=jax.ShapeDtypeStruct((B, S, 10), jnp.float32),
    )(u3, v2, b2, c2t, a3s, c3r)


def kernel(full, W1, b1, W2, b2, A1, c1, A2, c2, A3, c3):
    B, S, F = full.shape
    K = max(1, int(S * _SPARSITY))
    KP = ((K + 15) // 16) * 16
    G = K // 4

    x = full.reshape(B * S, F)
    keys, u, vf, b2blk, c2t, a3s = _run_stage_a(
        x, W1.T, b1.reshape(1, 32), W2.T, A1, c1.reshape(1, 64),
        A2, c2.reshape(1, 32), A3)

    sc_select = _make_sc_select(S, K, KP)
    v = sc_select(keys.reshape(B * S), vf)

    u3 = u.reshape(B, S, 64)
    v2 = v[:, :K, :64].reshape(B, G, 256)

    out = _run_stage_c(u3, v2, b2blk, c2t, a3s, c3.reshape(1, F), K, TS=1024)
    return out


# speedup vs baseline: 1.0226x; 1.0029x over previous
import dataclasses
import functools

import jax
import jax.numpy as jnp
from jax import lax
from jax.experimental import pallas as pl
from jax.experimental.pallas import tpu as pltpu
from jax.experimental.pallas import tpu_sc as plsc

_SPARSITY = 0.1


def _stage_a_body(F, x_ref, w1t_ref, b1_ref, w2t_ref, a1_ref, c1_ref,
                  a2_ref, c2_ref, a3_ref,
                  keys_ref, u_ref, vf_ref, b2_ref, c2t_ref, a3s_ref):
    x = x_ref[...]
    h = jax.nn.relu(jnp.dot(x, w1t_ref[...]) + b1_ref[...])
    sc = jnp.dot(h, w2t_ref[...])
    b = jax.lax.bitcast_convert_type(sc, jnp.int32)
    neg = jax.lax.shift_right_arithmetic(b, 31)
    ukey = b ^ (neg | jnp.int32(-2147483648))
    n = ukey.shape[0]
    keys_ref[...] = ukey.reshape(n // 128, 128)
    a1 = a1_ref[...]
    hiprec = jax.lax.Precision.HIGHEST
    u_ref[...] = jax.lax.dot_general(
        x, a1[:, :F], (((1,), (1,)), ((), ())),
        precision=hiprec)
    a1kv = a1[:, F:2 * F] + a1[:, 2 * F:]
    vf = jax.lax.dot_general(
        x, a1kv, (((1,), (1,)), ((), ())),
        precision=hiprec) + c1_ref[...]
    vf_ref[...] = jnp.concatenate(
        [vf, jnp.zeros_like(vf)], axis=1)
    a2t = jnp.transpose(a2_ref[...])
    z = jnp.zeros_like(a2t)
    rows = [jnp.concatenate([a2t if q == p else z for q in range(4)],
                            axis=1) for p in range(4)]
    b2_ref[...] = jnp.concatenate(rows, axis=0).astype(jnp.bfloat16)
    c2r = c2_ref[...]
    c2t_ref[...] = jnp.concatenate([c2r] * 4, axis=1)
    a3t = jnp.transpose(a3_ref[...])
    a3s_ref[...] = jnp.concatenate([a3t] * 4, axis=0)


def _run_stage_a(x, W1T, b1, W2T, A1, c1, A2, c2, A3):
    n, F = x.shape
    return pl.pallas_call(
        functools.partial(_stage_a_body, F),
        out_shape=(
            jax.ShapeDtypeStruct((n // 128, 128), jnp.int32),
            jax.ShapeDtypeStruct((n, 64), jnp.float32),
            jax.ShapeDtypeStruct((n, 128), jnp.float32),
            jax.ShapeDtypeStruct((256, 128), jnp.bfloat16),
            jax.ShapeDtypeStruct((1, 128), jnp.float32),
            jax.ShapeDtypeStruct((128, F), jnp.float32),
        ),
    )(x, W1T, b1, W2T, A1, c1, A2, c2, A3)


def _make_sc_select(S, K, KP):
    n_vregs = S // 16
    n_rows = KP // 16
    mesh = plsc.VectorSubcoreMesh(core_axis_name="c", subcore_axis_name="s")

    def body(keys_hbm, vf_hbm, out_hbm, key_v, idx_v, vbuf, hist_v, sem):
        cid = lax.axis_index("c")
        sid = lax.axis_index("s")

        @pl.when(sid == 0)
        def _():
            base = cid * S
            pltpu.sync_copy(keys_hbm.at[pl.ds(base, S)], key_v)

            iot = lax.iota(jnp.int32, 16)
            ones = jnp.zeros((16,), jnp.int32) + 1

            prefix = jnp.int32(0)
            k_rem = jnp.int32(K)
            for p in range(4):
                shift = 24 - 8 * p
                for r in range(16):
                    hist_v[r] = jnp.zeros((16,), jnp.int32)

                pfx = prefix

                @pl.loop(0, n_vregs)
                def _(i, shift=shift, p=p, pfx=pfx):
                    kk = key_v[pl.ds(i * 16, 16)]
                    byte = lax.shift_right_logical(kk, shift) & jnp.int32(255)
                    row = lax.shift_right_logical(byte, 4)
                    col = byte & jnp.int32(15)
                    if p == 0:
                        plsc.addupdate_scatter(hist_v, [row, col], ones)
                    else:
                        act = lax.shift_right_logical(
                            kk, shift + 8) == pfx
                        plsc.addupdate_scatter(hist_v, [row, col], ones,
                                               mask=act)

                carry = jnp.int32(0)
                best = jnp.int32(0)
                for r in range(15, -1, -1):
                    h = hist_v[r]
                    tot = jnp.sum(h)
                    incl = plsc.cumsum(h)
                    ge = (carry + tot) - incl + h
                    cand = jnp.max(jnp.where(ge >= k_rem,
                                             iot + 16 * r, 0))
                    best = jnp.maximum(best, cand)
                    carry = carry + tot
                bsel = best
                accv = jnp.zeros((16,), jnp.int32)
                for r in range(16):
                    accv = accv + jnp.where(iot + 16 * r > bsel,
                                            hist_v[r], 0)
                cnt_gt = jnp.sum(accv)
                prefix = prefix * 256 + bsel
                k_rem = k_rem - cnt_gt

            utarget = prefix
            need = k_rem
            ngt = K - need
            sgn = jnp.int32(-2147483648)
            tvx = utarget ^ sgn

            for r in range(n_rows):
                idx_v[r] = jnp.zeros((16,), jnp.int32) + base

            def sbody(i, packed):
                cgt = packed // 8192
                ceq = packed - cgt * 8192
                kk = key_v[pl.ds(i * 16, 16)]
                gt = (kk ^ sgn) > tvx
                eq = kk == utarget
                gti = jnp.where(gt, 1, 0).astype(jnp.int32)
                eqi = jnp.where(eq, 1, 0).astype(jnp.int32)
                pgt = plsc.cumsum(gti) + cgt
                peq = plsc.cumsum(eqi) + ceq
                slot = jnp.where(gt, pgt - 1, ngt + peq - 1)
                slot = jnp.minimum(slot, jnp.int32(KP - 1))
                slot = jnp.maximum(slot, jnp.int32(0))
                m = gt | (eq & (peq <= need))
                tok = lax.iota(jnp.int32, 16) + (i * 16 + base)
                row = lax.shift_right_arithmetic(slot, 4)
                col = slot & jnp.int32(15)
                plsc.store_scatter(idx_v, [row, col], tok, mask=m)
                return ((cgt + jnp.sum(gti)) * 8192 + (ceq + jnp.sum(eqi)))

            lax.fori_loop(0, n_vregs, sbody, jnp.int32(0))

            copies = []
            for r in range(n_rows):
                copies.append(pltpu.async_copy(
                    vf_hbm.at[idx_v.at[r]],
                    vbuf.at[pl.ds(r * 16, 16)], sem))
            for c in copies:
                c.wait()
            pltpu.sync_copy(vbuf, out_hbm.at[cid])

    cp = pltpu.CompilerParams()
    if "needs_layout_passes" in pltpu.CompilerParams.__dataclass_fields__:
        cp = dataclasses.replace(cp, needs_layout_passes=False)
    return functools.partial(
        pl.kernel, body,
        out_type=jax.ShapeDtypeStruct((2, KP, 128), jnp.float32),
        mesh=mesh,
        compiler_params=cp,
        scratch_types=[
            pltpu.VMEM((S,), jnp.int32),
            pltpu.VMEM((KP // 16, 16), jnp.int32),
            pltpu.VMEM((KP, 128), jnp.float32),
            pltpu.VMEM((16, 16), jnp.int32),
            pltpu.SemaphoreType.DMA,
        ],
    )()


def _stage_c_body(groups, inv_k, u_ref, v2_ref, b2_ref, c2_ref, a3_ref,
                  c3_ref, out_ref):
    u = u_ref[0]
    u4 = jnp.concatenate([u, u, u, u], axis=1)
    v52 = v2_ref[0]
    b2b = b2_ref[...]
    c2t = c2_ref[...]
    ts = u.shape[0]
    acc = jnp.zeros((ts, 128), jnp.float32)
    for jg in range(groups):
        vrow = v52[jg:jg + 1, :]
        h = jax.nn.relu(u4 + vrow)
        g = jnp.dot(h.astype(jnp.bfloat16), b2b,
                    preferred_element_type=jnp.float32)
        acc = acc + jax.nn.relu(g + c2t)
    out = jax.lax.dot_general(
        acc * inv_k, a3_ref[...], (((1,), (0,)), ((), ())),
        precision=jax.lax.Precision.HIGHEST) + c3_ref[...]
    out_ref[0] = out


def _run_stage_c(u3, v2, b2, c2t, a3s, c3r, K, TS):
    B, S, _ = u3.shape
    G = v2.shape[1]
    body = functools.partial(_stage_c_body, G, 1.0 / K)
    return pl.pallas_call(
        body,
        grid=(B, S // TS),
        in_specs=[
            pl.BlockSpec((1, TS, 64), lambda b, i: (b, i, 0)),
            pl.BlockSpec((1, G, 256), lambda b, i: (b, 0, 0)),
            pl.BlockSpec((256, 128), lambda b, i: (0, 0)),
            pl.BlockSpec((1, 128), lambda b, i: (0, 0)),
            pl.BlockSpec((128, 10), lambda b, i: (0, 0)),
            pl.BlockSpec((1, 10), lambda b, i: (0, 0)),
        ],
        out_specs=pl.BlockSpec((1, TS, 10), lambda b, i: (b, i, 0)),
        out_shape=jax.ShapeDtypeStruct((B, S, 10), jnp.float32),
    )(u3, v2, b2, c2t, a3s, c3r)


def kernel(full, W1, b1, W2, b2, A1, c1, A2, c2, A3, c3):
    B, S, F = full.shape
    K = max(1, int(S * _SPARSITY))
    KP = ((K + 15) // 16) * 16
    G = K // 4

    x = full.reshape(B * S, F)
    keys, u, vf, b2blk, c2t, a3s = _run_stage_a(
        x, W1.T, b1.reshape(1, 32), W2.T, A1, c1.reshape(1, 64),
        A2, c2.reshape(1, 32), A3)

    sc_select = _make_sc_select(S, K, KP)
    v = sc_select(keys.reshape(B * S), vf)

    u3 = u.reshape(B, S, 64)
    v2 = v[:, :K, :64].reshape(B, G, 256)

    out = _run_stage_c(u3, v2, b2blk, c2t, a3s, c3.reshape(1, F), K, TS=2048)
    return out
